# baseline (speedup 1.0000x reference)
import functools

import jax
import jax.numpy as jnp
from jax import lax
from jax.experimental import pallas as pl
from jax.experimental.pallas import tpu as pltpu
from jax.experimental.pallas import tpu_sc as plsc

_NC = 2
_NS = 16
_NW = _NC * _NS

_G = 2
_NBUF = 4


def _embed_gather(table, idx2d):
    R, S = idx2d.shape
    V, D = table.shape
    assert R % _NW == 0
    r_per_w = R // _NW
    assert r_per_w % _G == 0
    n_grp = r_per_w // _G
    assert n_grp >= 2 * _NBUF and n_grp % _NBUF == 0
    mesh = plsc.VectorSubcoreMesh(core_axis_name="c", subcore_axis_name="s")

    @functools.partial(
        pl.kernel,
        mesh=mesh,
        out_type=jax.ShapeDtypeStruct((R, 56, 128), jnp.float32),
        scratch_types=[
            pltpu.VMEM((r_per_w, S), jnp.int32),
            pltpu.VMEM((_NBUF, _G, S, D), jnp.float32),
            pltpu.SemaphoreType.DMA((_NBUF,)),
            pltpu.SemaphoreType.DMA((_NBUF,)),
        ],
        compiler_params=pltpu.CompilerParams(use_tc_tiling_on_sc=False),
    )
    def k(table_hbm, idx_hbm, out_hbm, idx_v, rows_v, gsem, osem):
        wid = lax.axis_index("s") * _NC + lax.axis_index("c")
        base = wid * r_per_w
        pltpu.sync_copy(idx_hbm.at[pl.ds(base, r_per_w)], idx_v)

        def gather(grp, b):
            for t in range(_G):
                pltpu.async_copy(
                    table_hbm.at[idx_v.at[grp * _G + t]],
                    rows_v.at[b, t],
                    gsem.at[b],
                )

        def gather_wait(b):
            for t in range(_G):
                pltpu.make_async_copy(
                    table_hbm.at[idx_v.at[0]],
                    rows_v.at[b, t],
                    gsem.at[b],
                ).wait()

        def write(grp, b):
            pltpu.async_copy(
                rows_v.at[b],
                out_hbm.at[pl.ds(base + grp * _G, _G), pl.ds(0, S), pl.ds(0, D)],
                osem.at[b],
            )

        def write_wait(b):
            pltpu.make_async_copy(
                rows_v.at[b],
                out_hbm.at[pl.ds(base, _G), pl.ds(0, S), pl.ds(0, D)],
                osem.at[b],
            ).wait()

        for b in range(_NBUF):
            gather(b, b)

        def group(gi, carry):
            g = gi * _NBUF
            for b in range(_NBUF):
                gather_wait(b)
                write(g + b, b)
            for b in range(_NBUF):
                write_wait(b)
                gather(g + _NBUF + b, b)
            return carry

        lax.fori_loop(0, n_grp // _NBUF - 1, group, 0)

        g = n_grp - _NBUF
        for b in range(_NBUF):
            gather_wait(b)
            write(g + b, b)
        for b in range(_NBUF):
            write_wait(b)

    return k(table, idx2d)


def _slice_tc(x, S, D):
    R = x.shape[0]
    B = 128
    return pl.pallas_call(
        lambda xr, orf: orf.__setitem__(
            (slice(None),) * 3, xr[:, :S, :D]
        ),
        grid=(R // B,),
        in_specs=[pl.BlockSpec((B, x.shape[1], x.shape[2]), lambda i: (i, 0, 0))],
        out_specs=pl.BlockSpec((B, S, D), lambda i: (i, 0, 0)),
        out_shape=jax.ShapeDtypeStruct((R, S, D), jnp.float32),
    )(x)


def kernel(inputs, embedding):
    V, D = embedding.shape
    R, S = inputs.shape
    out = _embed_gather(embedding, inputs.astype(jnp.int32))
    return _slice_tc(out, S, D)


# speedup vs baseline: 1.6377x; 1.6377x over previous
import functools

import jax
import jax.numpy as jnp
from jax import lax
from jax.experimental import pallas as pl
from jax.experimental.pallas import tpu as pltpu
from jax.experimental.pallas import tpu_sc as plsc

_NC = 2
_NS = 16
_NW = _NC * _NS

_G = 4
_NBUF = 4


def _embed_gather(table, idx2d):
    R, S = idx2d.shape
    V, D = table.shape
    assert R % _NW == 0
    r_per_w = R // _NW
    assert r_per_w % _G == 0
    n_grp = r_per_w // _G
    assert n_grp >= 2 * _NBUF and n_grp % _NBUF == 0
    mesh = plsc.VectorSubcoreMesh(core_axis_name="c", subcore_axis_name="s")

    @functools.partial(
        pl.kernel,
        mesh=mesh,
        out_type=jax.ShapeDtypeStruct((R, 56, 128), jnp.float32),
        scratch_types=[
            pltpu.VMEM((r_per_w, S), jnp.int32),
            pltpu.VMEM((_NBUF, _G, S, D), jnp.float32),
            pltpu.SemaphoreType.DMA((_NBUF,)),
            pltpu.SemaphoreType.DMA((_NBUF,)),
        ],
        compiler_params=pltpu.CompilerParams(use_tc_tiling_on_sc=False),
    )
    def k(table_hbm, idx_hbm, out_hbm, idx_v, rows_v, gsem, osem):
        wid = lax.axis_index("s") * _NC + lax.axis_index("c")
        base = wid * r_per_w
        pltpu.sync_copy(idx_hbm.at[pl.ds(base, r_per_w)], idx_v)

        def gather(grp, b):
            for t in range(_G):
                pltpu.async_copy(
                    table_hbm.at[idx_v.at[grp * _G + t]],
                    rows_v.at[b, t],
                    gsem.at[b],
                )

        def gather_wait(b):
            for t in range(_G):
                pltpu.make_async_copy(
                    table_hbm.at[idx_v.at[0]],
                    rows_v.at[b, t],
                    gsem.at[b],
                ).wait()

        def write(grp, b):
            pltpu.async_copy(
                rows_v.at[b],
                out_hbm.at[pl.ds(base + grp * _G, _G), pl.ds(0, S), pl.ds(0, D)],
                osem.at[b],
            )

        def write_wait(b):
            pltpu.make_async_copy(
                rows_v.at[b],
                out_hbm.at[pl.ds(base, _G), pl.ds(0, S), pl.ds(0, D)],
                osem.at[b],
            ).wait()

        for b in range(_NBUF):
            gather(b, b)

        def group(gi, carry):
            g = gi * _NBUF
            for b in range(_NBUF):
                gather_wait(b)
                write(g + b, b)
            for b in range(_NBUF):
                write_wait(b)
                gather(g + _NBUF + b, b)
            return carry

        lax.fori_loop(0, n_grp // _NBUF - 1, group, 0)

        g = n_grp - _NBUF
        for b in range(_NBUF):
            gather_wait(b)
            write(g + b, b)
        for b in range(_NBUF):
            write_wait(b)

    return k(table, idx2d)


def kernel(inputs, embedding):
    V, D = embedding.shape
    R, S = inputs.shape
    out = _embed_gather(embedding, inputs.astype(jnp.int32))
    return out[:, :S, :D]
